# baseline (speedup 1.0000x reference)
import jax
import jax.numpy as jnp
from jax import lax
from jax.experimental import pallas as pl
from jax.experimental.pallas import tpu as pltpu
from jax.experimental.pallas import tpu_sc as plsc

VOCAB = 1000000
EMB_DIM = 64
NEG_NUM = 64
BATCH = 4096

NC = 2
NS = 16
NW = NC * NS
BPW = BATCH // NW

CHUNK = 2
NBUF = 2
NCHUNK = BPW // CHUNK


def _w2v_body(in0_hbm, in1_hbm, in2_hbm, embT_hbm, embw2_hbm, embb_hbm,
              outt_hbm, outn_hbm,
              idx0_v, idx1_v, idx2_v, idx1h_v, idx2h_v, par1_v,
              bbufs, inp_vf, tw_v, tb_v, w_bufs, nb_bufs, outt_v, outn_v,
              bsem, tsem, wsem, nbsem):
    wid = lax.axis_index("s") * NC + lax.axis_index("c")
    base = wid * BPW

    iota = lax.iota(jnp.int32, 16)
    cols = [jnp.full((16,), d, jnp.int32) for d in range(EMB_DIM)]

    pltpu.sync_copy(in0_hbm.at[pl.ds(base, BPW)], idx0_v)
    pltpu.sync_copy(in1_hbm.at[pl.ds(base, BPW)], idx1_v)
    pltpu.sync_copy(in2_hbm.at[pl.ds(wid * NCHUNK, NCHUNK), :], idx2_v)

    for j in range(BPW // 16):
        v = idx1_v[pl.ds(16 * j, 16)]
        idx1h_v[pl.ds(16 * j, 16)] = v // 2
        par1_v[pl.ds(16 * j, 16)] = (v % 2) * EMB_DIM

    def half_body(r, carry):
        for j in range(CHUNK * NEG_NUM // 16):
            idx2h_v[r, pl.ds(16 * j, 16)] = idx2_v[r, pl.ds(16 * j, 16)] // 2
        return carry

    lax.fori_loop(0, NCHUNK, half_body, 0)

    pltpu.async_copy(embw2_hbm.at[idx1h_v], tw_v, tsem)
    pltpu.async_copy(embb_hbm.at[idx1_v], tb_v, tsem)

    def start_chunk(c, p):
        pltpu.async_copy(embw2_hbm.at[idx2h_v.at[c]], w_bufs.at[p], wsem.at[p])
        pltpu.async_copy(embb_hbm.at[idx2_v.at[c]], nb_bufs.at[p], nbsem.at[p])

    def wait_chunk(c, p):
        pltpu.make_async_copy(
            embw2_hbm.at[idx2h_v.at[c]], w_bufs.at[p], wsem.at[p]).wait()
        pltpu.make_async_copy(
            embb_hbm.at[idx2_v.at[c]], nb_bufs.at[p], nbsem.at[p]).wait()

    for p in range(NBUF):
        start_chunk(p, p)

    def issue_blk(v, p):
        cb = (v // 128) * 128
        pltpu.async_copy(embT_hbm.at[:, pl.ds(cb, 128)], bbufs.at[p],
                         bsem.at[p])

    def wait_blk(v, p):
        cb = (v // 128) * 128
        pltpu.make_async_copy(embT_hbm.at[:, pl.ds(cb, 128)], bbufs.at[p],
                              bsem.at[p]).wait()

    first = idx0_v[pl.ds(0, 16)]
    issue_blk(first[0], 0)

    def inp_body(ci, carry):
        vec = idx0_v[pl.ds(ci * 16, 16)]
        nci = jnp.minimum(ci + 1, (BPW // 16) - 1)
        nvec = idx0_v[pl.ds(nci * 16, 16)]
        for lane in range(16):
            p = lane % 2
            v = vec[lane]
            if lane + 1 < 16:
                issue_blk(vec[lane + 1], 1 - p)
            else:
                @pl.when(ci + 1 < BPW // 16)
                def _():
                    issue_blk(nvec[0], 1 - p)
            wait_blk(v, p)
            col = jnp.full((16,), v % 128, jnp.int32)
            i = ci * 16 + lane
            for g in range(4):
                w = plsc.load_gather(bbufs.at[p], [iota + 16 * g, col])
                inp_vf[pl.ds(i * EMB_DIM + 16 * g, 16)] = w
        return carry

    lax.fori_loop(0, BPW // 16, inp_body, 0)

    pltpu.make_async_copy(embw2_hbm.at[idx1h_v], tw_v, tsem).wait()
    pltpu.make_async_copy(embb_hbm.at[idx1_v], tb_v, tsem).wait()

    def t_body(grp, carry):
        brows = iota + grp * 16
        brows64 = brows * EMB_DIM
        parc = par1_v[pl.ds(grp * 16, 16)]
        acc = tb_v[pl.ds(grp * 16, 16)]
        for d in range(EMB_DIM):
            a = plsc.load_gather(inp_vf, [brows64 + cols[d]])
            t = plsc.load_gather(tw_v, [brows, parc + cols[d]])
            acc = acc + a * t
        outt_v[pl.ds(grp * 16, 16)] = acc
        return carry

    lax.fori_loop(0, BPW // 16, t_body, 0)

    def n_body(i, carry):
        for p in range(NBUF):
            c = i * NBUF + p
            wait_chunk(c, p)
            for cb in range(CHUNK):
                b = c * CHUNK + cb
                inp_c = [inp_vf[pl.ds(b * EMB_DIM + 16 * h, 16)]
                         for h in range(4)]
                par_c = [idx2_v[c, pl.ds(cb * NEG_NUM + 16 * h, 16)]
                         for h in range(4)]
                accs = [nb_bufs[p, pl.ds(cb * NEG_NUM + 16 * g, 16)]
                        for g in range(4)]
                for k in range(NEG_NUM):
                    s = inp_c[k // 16][k % 16]
                    half = (par_c[k // 16][k % 16] % 2) * EMB_DIM
                    row = cb * NEG_NUM + k
                    for g in range(4):
                        accs[g] = accs[g] + w_bufs[
                            p, row, pl.ds(half + 16 * g, 16)] * s
                for g in range(4):
                    outn_v[pl.ds(b * NEG_NUM + 16 * g, 16)] = accs[g]

            @pl.when(c + NBUF < NCHUNK)
            def _():
                start_chunk(c + NBUF, p)
        return carry

    lax.fori_loop(0, NCHUNK // NBUF, n_body, 0)

    pltpu.sync_copy(outt_v, outt_hbm.at[pl.ds(base, BPW)])
    pltpu.sync_copy(outn_v, outn_hbm.at[pl.ds(base * NEG_NUM, BPW * NEG_NUM)])


@jax.jit
def _w2v(in0, in1, in2, embT, emb_w2, emb_b):
    mesh = plsc.VectorSubcoreMesh(core_axis_name="c", subcore_axis_name="s")
    f = pl.kernel(
        _w2v_body,
        out_type=(
            jax.ShapeDtypeStruct((BATCH,), jnp.float32),
            jax.ShapeDtypeStruct((BATCH * NEG_NUM,), jnp.float32),
        ),
        mesh=mesh,
        compiler_params=pltpu.CompilerParams(
            needs_layout_passes=False, use_tc_tiling_on_sc=True),
        scratch_types=[
            pltpu.VMEM((BPW,), jnp.int32),
            pltpu.VMEM((BPW,), jnp.int32),
            pltpu.VMEM((NCHUNK, CHUNK * NEG_NUM), jnp.int32),
            pltpu.VMEM((BPW,), jnp.int32),
            pltpu.VMEM((NCHUNK, CHUNK * NEG_NUM), jnp.int32),
            pltpu.VMEM((BPW,), jnp.int32),
            pltpu.VMEM((2, EMB_DIM, 128), jnp.float32),
            pltpu.VMEM((BPW * EMB_DIM,), jnp.float32),
            pltpu.VMEM((BPW, 2 * EMB_DIM), jnp.float32),
            pltpu.VMEM((BPW,), jnp.float32),
            pltpu.VMEM((NBUF, CHUNK * NEG_NUM, 2 * EMB_DIM), jnp.float32),
            pltpu.VMEM((NBUF, CHUNK * NEG_NUM), jnp.float32),
            pltpu.VMEM((BPW,), jnp.float32),
            pltpu.VMEM((BPW * NEG_NUM,), jnp.float32),
            pltpu.SemaphoreType.DMA((2,)),
            pltpu.SemaphoreType.DMA,
            pltpu.SemaphoreType.DMA((NBUF,)),
            pltpu.SemaphoreType.DMA((NBUF,)),
        ],
    )
    return f(in0, in1, in2, embT, emb_w2, emb_b)


def kernel(inputs_0, inputs_1, inputs_2, emb, emb_w, emb_b):
    in0 = inputs_0.reshape(BATCH).astype(jnp.int32)
    in1 = inputs_1.reshape(BATCH).astype(jnp.int32)
    in2 = inputs_2.astype(jnp.int32).reshape(BATCH // CHUNK, CHUNK * NEG_NUM)
    true_flat, neg_flat = _w2v(in0, in1, in2, emb.T,
                               emb_w.reshape(VOCAB // 2, 2 * EMB_DIM),
                               emb_b.reshape(VOCAB))
    return true_flat.reshape(BATCH, 1), neg_flat.reshape(BATCH, NEG_NUM)
